# baseline (speedup 1.0000x reference)
import functools

import jax
import jax.numpy as jnp
from jax import lax
from jax.experimental import pallas as pl
from jax.experimental.pallas import tpu as pltpu
from jax.experimental.pallas import tpu_sc as plsc

N = 10000
D = 128
DH = D // 2
E = 320000
EP = E + N
NEG = 0.2

NT = 16
B = 128
NB = 162
C_PER = NB * B
EPAD = NT * C_PER
NPAD = 10240
SLICE = NPAD // NT

BR = 512
GRID = NPAD // BR


def _mm_alpha_body(x_ref, w_ref, as_ref, ad_ref, h_ref, s_ref, d_ref):
    h = jnp.dot(x_ref[...], w_ref[...], preferred_element_type=jnp.float32)
    h_ref[0] = h[:, :DH]
    h_ref[1] = h[:, DH:]
    s_ref[...] = jnp.sum(h * as_ref[...][None, :], axis=1)
    d_ref[...] = jnp.sum(h * ad_ref[...][None, :], axis=1)
    @pl.when(pl.program_id(0) == GRID - 1)
    def _():
        idx = lax.broadcasted_iota(jnp.int32, (BR,), 0)
        s_ref[...] = jnp.where(idx == BR - 1, -1e30, s_ref[...])
        d_ref[...] = jnp.where(idx == BR - 1, 0.0, d_ref[...])


def _mm_alpha(xp, W, a_s, a_d):
    return pl.pallas_call(
        _mm_alpha_body,
        grid=(GRID,),
        in_specs=[
            pl.BlockSpec((BR, D), lambda i: (i, 0)),
            pl.BlockSpec((D, D), lambda i: (0, 0)),
            pl.BlockSpec((D,), lambda i: (0,)),
            pl.BlockSpec((D,), lambda i: (0,)),
        ],
        out_specs=[
            pl.BlockSpec((2, BR, DH), lambda i: (0, i, 0)),
            pl.BlockSpec((BR,), lambda i: (i,)),
            pl.BlockSpec((BR,), lambda i: (i,)),
        ],
        out_shape=[
            jax.ShapeDtypeStruct((2, NPAD, DH), jnp.float32),
            jax.ShapeDtypeStruct((NPAD,), jnp.float32),
            jax.ShapeDtypeStruct((NPAD,), jnp.float32),
        ],
    )(xp, W, a_s, a_d)


def _combine(acc_ref, den_ref, b_ref):
    a = jnp.concatenate([acc_ref[0], acc_ref[1]], axis=1)
    dn = jnp.sum(den_ref[...], axis=0)
    dn = jnp.where(dn > 0, dn, 1.0)
    return a / dn[:, None] + b_ref[...][None, :]


def _ep_mm_body(acc_ref, den_ref, b_ref, w_ref, as_ref, ad_ref,
                h_ref, s_ref, d_ref):
    z = _combine(acc_ref, den_ref, b_ref)
    z = jnp.where(z > 0, z, jnp.exp(z) - 1.0)
    h = jnp.dot(z, w_ref[...], preferred_element_type=jnp.float32)
    h_ref[0] = h[:, :DH]
    h_ref[1] = h[:, DH:]
    s_ref[...] = jnp.sum(h * as_ref[...][None, :], axis=1)
    d_ref[...] = jnp.sum(h * ad_ref[...][None, :], axis=1)
    @pl.when(pl.program_id(0) == GRID - 1)
    def _():
        idx = lax.broadcasted_iota(jnp.int32, (BR,), 0)
        s_ref[...] = jnp.where(idx == BR - 1, -1e30, s_ref[...])
        d_ref[...] = jnp.where(idx == BR - 1, 0.0, d_ref[...])


def _ep_mm(acc, den, b, W, a_s, a_d):
    return pl.pallas_call(
        _ep_mm_body,
        grid=(GRID,),
        in_specs=[
            pl.BlockSpec((2, BR, DH), lambda i: (0, i, 0)),
            pl.BlockSpec((NT, BR), lambda i: (0, i)),
            pl.BlockSpec((D,), lambda i: (0,)),
            pl.BlockSpec((D, D), lambda i: (0, 0)),
            pl.BlockSpec((D,), lambda i: (0,)),
            pl.BlockSpec((D,), lambda i: (0,)),
        ],
        out_specs=[
            pl.BlockSpec((2, BR, DH), lambda i: (0, i, 0)),
            pl.BlockSpec((BR,), lambda i: (i,)),
            pl.BlockSpec((BR,), lambda i: (i,)),
        ],
        out_shape=[
            jax.ShapeDtypeStruct((2, NPAD, DH), jnp.float32),
            jax.ShapeDtypeStruct((NPAD,), jnp.float32),
            jax.ShapeDtypeStruct((NPAD,), jnp.float32),
        ],
    )(acc, den, b, W, a_s, a_d)


def _final_body(acc_ref, den_ref, b_ref, o_ref):
    o_ref[...] = _combine(acc_ref, den_ref, b_ref)


def _final(acc, den, b):
    return pl.pallas_call(
        _final_body,
        grid=(GRID,),
        in_specs=[
            pl.BlockSpec((2, BR, DH), lambda i: (0, i, 0)),
            pl.BlockSpec((NT, BR), lambda i: (0, i)),
            pl.BlockSpec((D,), lambda i: (0,)),
        ],
        out_specs=pl.BlockSpec((BR, D), lambda i: (i, 0)),
        out_shape=jax.ShapeDtypeStruct((NPAD, D), jnp.float32),
    )(acc, den, b)


def _edge_pass_body(src_hbm, dst_hbm, asrc_hbm, adst_hbm, h_hbm,
                    acc_out, den_out,
                    src_v, dst_v, asrc_v, adst_v, pbatch, den_t,
                    rows0, rows1,
                    acc_sh, gsem, ssem):
    c = lax.axis_index("c")
    s = lax.axis_index("s")
    base_row = s * SLICE

    zero16 = jnp.zeros((16,), jnp.float32)

    def zrow(i, carry):
        for q in range(DH // 16):
            rows0[i, pl.ds(q * 16, 16)] = zero16
        return carry

    lax.fori_loop(0, B, zrow, 0)

    def zden(i, carry):
        den_t[pl.ds(i * 16, 16)] = zero16
        return carry

    lax.fori_loop(0, NPAD // 16, zden, 0)

    for r in range(SLICE // B):
        pltpu.sync_copy(rows0, acc_sh.at[pl.ds(base_row + r * B, B)])

    pltpu.sync_copy(src_hbm.at[s], src_v)
    pltpu.sync_copy(dst_hbm.at[s], dst_v)
    pltpu.sync_copy(asrc_hbm, asrc_v)
    pltpu.sync_copy(adst_hbm, adst_v)

    plsc.subcore_barrier()

    pltpu.async_copy(h_hbm.at[c].at[src_v.at[0]], rows0, gsem)

    def outer_body(ob, carry):
        for par in range(2):
            rowsb = rows0 if par == 0 else rows1
            rowsn = rows1 if par == 0 else rows0
            b = ob * 2 + par

            pltpu.make_async_copy(
                h_hbm.at[c].at[src_v.at[b]], rowsb, gsem).wait()

            @pl.when(b >= 1)
            def _():
                pltpu.make_async_copy(
                    rowsn, acc_sh.at[dst_v.at[b]], ssem).wait()

            @pl.when(b + 1 < NB)
            def _():
                pltpu.async_copy(h_hbm.at[c].at[src_v.at[b + 1]], rowsn, gsem)

            def group_body(g, carry2):
                gbase = g * 16
                sv = src_v[b, pl.ds(gbase, 16)]
                dv = dst_v[b, pl.ds(gbase, 16)]
                a1 = plsc.load_gather(asrc_v, [sv])
                a2 = plsc.load_gather(adst_v, [dv])
                e = a1 + a2
                e = jnp.where(e > 0, e, e * NEG)
                p = jnp.exp(e)
                pbatch[pl.ds(gbase, 16)] = p
                plsc.addupdate_scatter(den_t, [dv], p)
                for ee in range(16):
                    i = gbase + ee
                    pv = plsc.load_gather(
                        pbatch, [jnp.full((16,), i, jnp.int32)])
                    for j in range(DH // 16):
                        rowsb[i, pl.ds(j * 16, 16)] = (
                            rowsb[i, pl.ds(j * 16, 16)] * pv)
                return carry2

            lax.fori_loop(0, B // 16, group_body, 0)

            pltpu.async_copy(rowsb, acc_sh.at[dst_v.at[b]], ssem, add=True)

        return carry

    lax.fori_loop(0, NB // 2, outer_body, 0)

    pltpu.make_async_copy(rows1, acc_sh.at[dst_v.at[0]], ssem).wait()

    plsc.subcore_barrier()

    pltpu.sync_copy(acc_sh.at[pl.ds(base_row, SLICE)],
                    acc_out.at[c, pl.ds(base_row, SLICE)])

    @pl.when(c == 0)
    def _():
        pltpu.sync_copy(den_t, den_out.at[s])


@functools.lru_cache(maxsize=1)
def _build_edge_pass():
    mesh = plsc.VectorSubcoreMesh(core_axis_name="c", subcore_axis_name="s")
    return pl.kernel(
        _edge_pass_body,
        mesh=mesh,
        compiler_params=pltpu.CompilerParams(
            needs_layout_passes=False, use_tc_tiling_on_sc=False),
        out_type=[
            jax.ShapeDtypeStruct((2, NPAD, DH), jnp.float32),
            jax.ShapeDtypeStruct((NT, NPAD), jnp.float32),
        ],
        scratch_types=[
            pltpu.VMEM((NB, B), jnp.int32),
            pltpu.VMEM((NB, B), jnp.int32),
            pltpu.VMEM((NPAD,), jnp.float32),
            pltpu.VMEM((NPAD,), jnp.float32),
            pltpu.VMEM((B,), jnp.float32),
            pltpu.VMEM((NPAD,), jnp.float32),
            pltpu.VMEM((B, DH), jnp.float32),
            pltpu.VMEM((B, DH), jnp.float32),
            pltpu.VMEM_SHARED((NPAD, DH), jnp.float32),
            pltpu.SemaphoreType.DMA,
            pltpu.SemaphoreType.DMA,
        ],
    )


def kernel(x, edge_index, edge_weight, W1, a_s1, a_d1, b1, W2, a_s2, a_d2, b2):
    src = edge_index[0].astype(jnp.int32)
    dst = edge_index[1].astype(jnp.int32)
    loops = jnp.arange(N, dtype=jnp.int32)
    pad = jnp.full((EPAD - EP,), NPAD - 1, jnp.int32)
    srcp = jnp.concatenate([src, loops, pad]).reshape(NT, NB, B)
    dstp = jnp.concatenate([dst, loops, pad]).reshape(NT, NB, B)
    xp = jnp.pad(x, ((0, NPAD - N), (0, 0)))

    edge_pass = _build_edge_pass()
    h1, s1, d1 = _mm_alpha(xp, W1, a_s1, a_d1)
    acc1, den1 = edge_pass(srcp, dstp, s1, d1, h1)
    h2, s2, d2 = _ep_mm(acc1, den1, b1, W2, a_s2, a_d2)
    acc2, den2 = edge_pass(srcp, dstp, s2, d2, h2)
    outp = _final(acc2, den2, b2)
    return outp[:N]


# speedup vs baseline: 1.6780x; 1.6780x over previous
import functools

import jax
import jax.numpy as jnp
from jax import lax
from jax.experimental import pallas as pl
from jax.experimental.pallas import tpu as pltpu
from jax.experimental.pallas import tpu_sc as plsc

N = 10000
D = 128
DH = D // 2
E = 320000
EP = E + N
NEG = 0.2

NT = 16
B = 128
NB = 162
C_PER = NB * B
EPAD = NT * C_PER
NPAD = 10240
SLICE = NPAD // NT

BR = 512
GRID = NPAD // BR


def _mm_alpha_body(x_ref, w_ref, as_ref, ad_ref, h_ref, s_ref, d_ref):
    h = jnp.dot(x_ref[...], w_ref[...], preferred_element_type=jnp.float32)
    h_ref[0] = h[:, :DH]
    h_ref[1] = h[:, DH:]
    s_ref[...] = jnp.sum(h * as_ref[...][None, :], axis=1)
    d_ref[...] = jnp.sum(h * ad_ref[...][None, :], axis=1)


def _mm_alpha(xp, W, a_s, a_d):
    return pl.pallas_call(
        _mm_alpha_body,
        grid=(GRID,),
        in_specs=[
            pl.BlockSpec((BR, D), lambda i: (i, 0)),
            pl.BlockSpec((D, D), lambda i: (0, 0)),
            pl.BlockSpec((D,), lambda i: (0,)),
            pl.BlockSpec((D,), lambda i: (0,)),
        ],
        out_specs=[
            pl.BlockSpec((2, BR, DH), lambda i: (0, i, 0)),
            pl.BlockSpec((BR,), lambda i: (i,)),
            pl.BlockSpec((BR,), lambda i: (i,)),
        ],
        out_shape=[
            jax.ShapeDtypeStruct((2, NPAD, DH), jnp.float32),
            jax.ShapeDtypeStruct((NPAD,), jnp.float32),
            jax.ShapeDtypeStruct((NPAD,), jnp.float32),
        ],
    )(xp, W, a_s, a_d)


def _combine(acc_ref, den_ref, b_ref):
    a = jnp.concatenate([acc_ref[0], acc_ref[1]], axis=1)
    dn = jnp.sum(den_ref[...], axis=0)
    dn = jnp.where(dn > 0, dn, 1.0)
    return a / dn[:, None] + b_ref[...][None, :]


def _ep_mm_body(acc_ref, den_ref, b_ref, w_ref, as_ref, ad_ref,
                h_ref, s_ref, d_ref):
    z = _combine(acc_ref, den_ref, b_ref)
    z = jnp.where(z > 0, z, jnp.exp(z) - 1.0)
    h = jnp.dot(z, w_ref[...], preferred_element_type=jnp.float32)
    h_ref[0] = h[:, :DH]
    h_ref[1] = h[:, DH:]
    s_ref[...] = jnp.sum(h * as_ref[...][None, :], axis=1)
    d_ref[...] = jnp.sum(h * ad_ref[...][None, :], axis=1)


def _ep_mm(acc, den, b, W, a_s, a_d):
    return pl.pallas_call(
        _ep_mm_body,
        grid=(GRID,),
        in_specs=[
            pl.BlockSpec((2, BR, DH), lambda i: (0, i, 0)),
            pl.BlockSpec((NT, BR), lambda i: (0, i)),
            pl.BlockSpec((D,), lambda i: (0,)),
            pl.BlockSpec((D, D), lambda i: (0, 0)),
            pl.BlockSpec((D,), lambda i: (0,)),
            pl.BlockSpec((D,), lambda i: (0,)),
        ],
        out_specs=[
            pl.BlockSpec((2, BR, DH), lambda i: (0, i, 0)),
            pl.BlockSpec((BR,), lambda i: (i,)),
            pl.BlockSpec((BR,), lambda i: (i,)),
        ],
        out_shape=[
            jax.ShapeDtypeStruct((2, NPAD, DH), jnp.float32),
            jax.ShapeDtypeStruct((NPAD,), jnp.float32),
            jax.ShapeDtypeStruct((NPAD,), jnp.float32),
        ],
    )(acc, den, b, W, a_s, a_d)


def _final_body(acc_ref, den_ref, b_ref, o_ref):
    o_ref[...] = _combine(acc_ref, den_ref, b_ref)


def _final(acc, den, b):
    return pl.pallas_call(
        _final_body,
        grid=(GRID,),
        in_specs=[
            pl.BlockSpec((2, BR, DH), lambda i: (0, i, 0)),
            pl.BlockSpec((NT, BR), lambda i: (0, i)),
            pl.BlockSpec((D,), lambda i: (0,)),
        ],
        out_specs=pl.BlockSpec((BR, D), lambda i: (i, 0)),
        out_shape=jax.ShapeDtypeStruct((NPAD, D), jnp.float32),
    )(acc, den, b)


def _edge_pass_body(src_hbm, dst_hbm, asrc_hbm, adst_hbm, h_hbm,
                    acc_out, den_out,
                    src_v, dst_v, asrc_v, adst_v, pbatch, den_t,
                    rows0, rows1,
                    acc_sh, gsem, ssem):
    c = lax.axis_index("c")
    s = lax.axis_index("s")
    base_row = s * SLICE

    zero16 = jnp.zeros((16,), jnp.float32)

    def zrow(i, carry):
        for q in range(DH // 16):
            rows0[i, pl.ds(q * 16, 16)] = zero16
        return carry

    lax.fori_loop(0, B, zrow, 0)

    def zden(i, carry):
        den_t[pl.ds(i * 16, 16)] = zero16
        return carry

    lax.fori_loop(0, NPAD // 16, zden, 0)

    for r in range(SLICE // B):
        pltpu.sync_copy(rows0, acc_sh.at[pl.ds(base_row + r * B, B)])

    pltpu.sync_copy(src_hbm.at[s], src_v)
    pltpu.sync_copy(dst_hbm.at[s], dst_v)
    pltpu.sync_copy(asrc_hbm, asrc_v)
    pltpu.sync_copy(adst_hbm, adst_v)

    plsc.subcore_barrier()

    base_eid = s * C_PER

    pltpu.async_copy(h_hbm.at[c].at[src_v.at[0]], rows0, gsem)

    def outer_body(ob, carry):
        for par in range(2):
            rowsb = rows0 if par == 0 else rows1
            rowsn = rows1 if par == 0 else rows0
            b = ob * 2 + par

            def pvec_body(i, carry2):
                sv = src_v[b, pl.ds(i * 16, 16)]
                dv = dst_v[b, pl.ds(i * 16, 16)]
                a1 = plsc.load_gather(asrc_v, [sv])
                a2 = plsc.load_gather(adst_v, [dv])
                e = a1 + a2
                e = jnp.where(e > 0, e, e * NEG)
                p = jnp.exp(e)
                eid = base_eid + b * B + i * 16 + lax.iota(jnp.int32, 16)
                p = jnp.where(eid < EP, p, 0.0)
                pbatch[pl.ds(i * 16, 16)] = p
                plsc.addupdate_scatter(den_t, [dv], p)
                return carry2

            lax.fori_loop(0, 8, pvec_body, 0, unroll=True)

            pltpu.make_async_copy(
                h_hbm.at[c].at[src_v.at[b]], rowsb, gsem).wait()

            @pl.when(b >= 1)
            def _():
                pltpu.make_async_copy(
                    rowsn, acc_sh.at[dst_v.at[b]], ssem).wait()

            @pl.when(b + 1 < NB)
            def _():
                pltpu.async_copy(h_hbm.at[c].at[src_v.at[b + 1]], rowsn, gsem)

            def scale_body(i, carry2):
                pv = plsc.load_gather(pbatch, [jnp.full((16,), i, jnp.int32)])
                for j in range(DH // 16):
                    rowsb[i, pl.ds(j * 16, 16)] = (
                        rowsb[i, pl.ds(j * 16, 16)] * pv)
                return carry2

            lax.fori_loop(0, B, scale_body, 0, unroll=4)

            pltpu.async_copy(rowsb, acc_sh.at[dst_v.at[b]], ssem, add=True)

        return carry

    lax.fori_loop(0, NB // 2, outer_body, 0)

    pltpu.make_async_copy(rows1, acc_sh.at[dst_v.at[0]], ssem).wait()

    plsc.subcore_barrier()

    pltpu.sync_copy(acc_sh.at[pl.ds(base_row, SLICE)],
                    acc_out.at[c, pl.ds(base_row, SLICE)])

    @pl.when(c == 0)
    def _():
        pltpu.sync_copy(den_t, den_out.at[s])


@functools.lru_cache(maxsize=1)
def _build_edge_pass():
    mesh = plsc.VectorSubcoreMesh(core_axis_name="c", subcore_axis_name="s")
    return pl.kernel(
        _edge_pass_body,
        mesh=mesh,
        compiler_params=pltpu.CompilerParams(
            needs_layout_passes=False, use_tc_tiling_on_sc=False),
        out_type=[
            jax.ShapeDtypeStruct((2, NPAD, DH), jnp.float32),
            jax.ShapeDtypeStruct((NT, NPAD), jnp.float32),
        ],
        scratch_types=[
            pltpu.VMEM((NB, B), jnp.int32),
            pltpu.VMEM((NB, B), jnp.int32),
            pltpu.VMEM((NPAD,), jnp.float32),
            pltpu.VMEM((NPAD,), jnp.float32),
            pltpu.VMEM((B,), jnp.float32),
            pltpu.VMEM((NPAD,), jnp.float32),
            pltpu.VMEM((B, DH), jnp.float32),
            pltpu.VMEM((B, DH), jnp.float32),
            pltpu.VMEM_SHARED((NPAD, DH), jnp.float32),
            pltpu.SemaphoreType.DMA,
            pltpu.SemaphoreType.DMA,
        ],
    )


def kernel(x, edge_index, edge_weight, W1, a_s1, a_d1, b1, W2, a_s2, a_d2, b2):
    src = edge_index[0].astype(jnp.int32)
    dst = edge_index[1].astype(jnp.int32)
    loops = jnp.arange(N, dtype=jnp.int32)
    pad = jnp.zeros((EPAD - EP,), jnp.int32)
    srcp = jnp.concatenate([src, loops, pad]).reshape(NT, NB, B)
    dstp = jnp.concatenate([dst, loops, pad]).reshape(NT, NB, B)
    xp = jnp.pad(x, ((0, NPAD - N), (0, 0)))

    edge_pass = _build_edge_pass()
    h1, s1, d1 = _mm_alpha(xp, W1, a_s1, a_d1)
    acc1, den1 = edge_pass(srcp, dstp, s1, d1, h1)
    h2, s2, d2 = _ep_mm(acc1, den1, b1, W2, a_s2, a_d2)
    acc2, den2 = edge_pass(srcp, dstp, s2, d2, h2)
    outp = _final(acc2, den2, b2)
    return outp[:N]
